# Initial kernel scaffold; baseline (speedup 1.0000x reference)
import jax
import jax.numpy as jnp
from jax.experimental import pallas as pl


def kernel(x, edge_index, feature_logits, W1, b1, W2, b2):
    raise NotImplementedError("write your pallas kernel here")


# speedup vs baseline: 14.5959x; 14.5959x over previous
import functools

import jax
import jax.numpy as jnp
from jax import lax
from jax.experimental import pallas as pl
from jax.experimental.pallas import tpu as pltpu
from jax.experimental.pallas import tpu_sc as plsc

N = 10000
E = 320000
F = 128
H = 256
C = 64
K = 64

NC = 2
NS = 16
NW = NC * NS
CH = 128
CH_PER_W = 79
EPW = CH_PER_W * CH
E_PAD = NW * EPW
N_ACC = N + 112
RPT = N_ACC // NS
DW = 8


@functools.cache
def _make_deg_kernel():
    mesh = plsc.VectorSubcoreMesh(core_axis_name="c", subcore_axis_name="s")

    @functools.partial(
        pl.kernel,
        mesh=mesh,
        out_type=jax.ShapeDtypeStruct((NC, N_ACC, DW), jnp.float32),
        scratch_types=[
            pltpu.VMEM((CH,), jnp.int32),
            pltpu.VMEM((CH, DW), jnp.float32),
            pltpu.VMEM_SHARED((N_ACC, DW), jnp.float32),
        ],
        compiler_params=pltpu.CompilerParams(use_tc_tiling_on_sc=False),
    )
    def _deg_kernel(dst_hbm, ones_hbm, zeros_hbm, out_hbm, didx, ones_v, acc):
        c = lax.axis_index("c")
        s = lax.axis_index("s")
        wid = s * NC + c
        pltpu.sync_copy(zeros_hbm.at[pl.ds(s * RPT, RPT)],
                        acc.at[pl.ds(s * RPT, RPT)])
        pltpu.sync_copy(ones_hbm, ones_v)
        plsc.subcore_barrier()

        def body(i, carry):
            eoff = wid * EPW + i * CH
            pltpu.sync_copy(dst_hbm.at[pl.ds(eoff, CH)], didx)
            pltpu.sync_copy(ones_v, acc.at[didx], add=True)
            return carry

        lax.fori_loop(0, CH_PER_W, body, 0)
        plsc.subcore_barrier()
        pltpu.sync_copy(acc.at[pl.ds(s * RPT, RPT)],
                        out_hbm.at[c, pl.ds(s * RPT, RPT)])

    return _deg_kernel


@functools.cache
def _make_prop_kernel(width):
    mesh = plsc.VectorSubcoreMesh(core_axis_name="c", subcore_axis_name="s")

    @functools.partial(
        pl.kernel,
        mesh=mesh,
        out_type=jax.ShapeDtypeStruct((NC, N_ACC, width), jnp.float32),
        scratch_types=[
            pltpu.VMEM((CH,), jnp.int32),
            pltpu.VMEM((CH,), jnp.int32),
            pltpu.VMEM((CH, width), jnp.float32),
            pltpu.VMEM_SHARED((N_ACC, width), jnp.float32),
        ],
        compiler_params=pltpu.CompilerParams(use_tc_tiling_on_sc=False),
    )
    def _prop_kernel(table_hbm, src_hbm, dst_hbm, zeros_hbm, out_hbm,
                     sidx, didx, rows, acc):
        c = lax.axis_index("c")
        s = lax.axis_index("s")
        wid = s * NC + c
        pltpu.sync_copy(zeros_hbm.at[pl.ds(s * RPT, RPT)],
                        acc.at[pl.ds(s * RPT, RPT)])
        plsc.subcore_barrier()

        def body(i, carry):
            eoff = wid * EPW + i * CH
            pltpu.sync_copy(src_hbm.at[pl.ds(eoff, CH)], sidx)
            pltpu.sync_copy(dst_hbm.at[pl.ds(eoff, CH)], didx)
            pltpu.sync_copy(table_hbm.at[sidx], rows)
            pltpu.sync_copy(rows, acc.at[didx], add=True)
            return carry

        lax.fori_loop(0, CH_PER_W, body, 0)
        plsc.subcore_barrier()
        pltpu.sync_copy(acc.at[pl.ds(s * RPT, RPT)],
                        out_hbm.at[c, pl.ds(s * RPT, RPT)])

    return _prop_kernel


def _compact_body(logits_ref, x_ref, w1_ref, xc_ref, w1c_ref):
    soft = jax.nn.sigmoid(logits_ref[...])
    a = soft[:, None]
    b = soft[None, :]
    i2 = lax.broadcasted_iota(jnp.int32, (F, F), 0)
    j2 = lax.broadcasted_iota(jnp.int32, (F, F), 1)
    beats = (b > a) | ((b == a) & (j2 < i2))
    rank = jnp.sum(beats.astype(jnp.int32), axis=1)
    sel = rank < K
    before = sel[None, :] & (j2 < i2)
    pos = jnp.sum(before.astype(jnp.int32), axis=1)
    kk = lax.broadcasted_iota(jnp.int32, (F, K), 1)
    P = (sel[:, None] & (pos[:, None] == kk)).astype(jnp.float32)
    xc_ref[...] = jnp.dot(x_ref[...], P, preferred_element_type=jnp.float32)
    w1c_ref[...] = lax.dot_general(
        P, w1_ref[...], (((0,), (0,)), ((), ())),
        preferred_element_type=jnp.float32)


def _scale_body(dc0_ref, dc1_ref, xc_ref, xt_ref, dinv_ref):
    deg = dc0_ref[...] + dc1_ref[...] + 1.0
    dinv = lax.rsqrt(jnp.maximum(deg, 1e-12))
    dinv_ref[...] = dinv
    xt_ref[...] = xc_ref[...] * dinv


def _mid_body(s1a_ref, s1b_ref, xt_ref, dinv_ref, w1c_ref, b1_ref, w2_ref,
              gt_ref):
    dinv = dinv_ref[...]
    xp = (s1a_ref[...] + s1b_ref[...] + xt_ref[...]) * dinv
    h = jnp.dot(xp, w1c_ref[...], preferred_element_type=jnp.float32)
    h = jnp.maximum(h + b1_ref[...], 0.0)
    g = jnp.dot(h, w2_ref[...], preferred_element_type=jnp.float32)
    gt_ref[...] = g * dinv


def _out_body(s2a_ref, s2b_ref, gt_ref, dinv_ref, b2_ref, out_ref):
    out_ref[...] = ((s2a_ref[...] + s2b_ref[...] + gt_ref[...])
                    * dinv_ref[...] + b2_ref[...])


_MB = 2000


def kernel(x, edge_index, feature_logits, W1, b1, W2, b2):
    src = edge_index[0]
    dst = edge_index[1]
    pad = E_PAD - E
    srcp = jnp.concatenate([src, jnp.zeros((pad,), jnp.int32)])
    dstp = jnp.concatenate([dst, jnp.full((pad,), N, jnp.int32)])
    zeros_w = jnp.zeros((N_ACC, K), jnp.float32)
    zeros_1 = jnp.zeros((N_ACC, DW), jnp.float32)
    ones_ch = jnp.ones((CH, DW), jnp.float32)

    dc = _make_deg_kernel()(dstp, ones_ch, zeros_1)

    xc, w1c = pl.pallas_call(
        _compact_body,
        out_shape=(jax.ShapeDtypeStruct((N, K), jnp.float32),
                   jax.ShapeDtypeStruct((K, H), jnp.float32)),
    )(feature_logits, x, W1)

    xt, dinv = pl.pallas_call(
        _scale_body,
        out_shape=(jax.ShapeDtypeStruct((N, K), jnp.float32),
                   jax.ShapeDtypeStruct((N, 1), jnp.float32)),
    )(dc[0, :N, :1], dc[1, :N, :1], xc)

    s1 = _make_prop_kernel(K)(xt, srcp, dstp, zeros_w)

    gt = pl.pallas_call(
        _mid_body,
        grid=(N // _MB,),
        in_specs=[
            pl.BlockSpec((_MB, K), lambda i: (i, 0)),
            pl.BlockSpec((_MB, K), lambda i: (i, 0)),
            pl.BlockSpec((_MB, K), lambda i: (i, 0)),
            pl.BlockSpec((_MB, 1), lambda i: (i, 0)),
            pl.BlockSpec((K, H), lambda i: (0, 0)),
            pl.BlockSpec((1, H), lambda i: (0, 0)),
            pl.BlockSpec((H, C), lambda i: (0, 0)),
        ],
        out_specs=pl.BlockSpec((_MB, C), lambda i: (i, 0)),
        out_shape=jax.ShapeDtypeStruct((N, C), jnp.float32),
    )(s1[0, :N], s1[1, :N], xt, dinv, w1c, b1.reshape(1, H), W2)

    s2 = _make_prop_kernel(C)(gt, srcp, dstp, zeros_w)

    out = pl.pallas_call(
        _out_body,
        out_shape=jax.ShapeDtypeStruct((N, C), jnp.float32),
    )(s2[0, :N], s2[1, :N], gt, dinv, b2.reshape(1, C))
    return out


# speedup vs baseline: 15.4689x; 1.0598x over previous
import functools

import jax
import jax.numpy as jnp
from jax import lax
from jax.experimental import pallas as pl
from jax.experimental.pallas import tpu as pltpu
from jax.experimental.pallas import tpu_sc as plsc

N = 10000
E = 320000
F = 128
H = 256
C = 64
K = 64

NC = 2
NS = 16
NW = NC * NS
CH = 128
CH_PER_W = 80
EPW = CH_PER_W * CH
E_PAD = NW * EPW
N_ACC = N + 112
RPT = N_ACC // NS
DW = 8
NBUF = 8
PF = 4
NGRP = CH_PER_W // NBUF


@functools.cache
def _make_deg_kernel():
    mesh = plsc.VectorSubcoreMesh(core_axis_name="c", subcore_axis_name="s")

    @functools.partial(
        pl.kernel,
        mesh=mesh,
        out_type=jax.ShapeDtypeStruct((NC, N_ACC, DW), jnp.float32),
        scratch_types=[
            pltpu.VMEM((CH_PER_W, CH), jnp.int32),
            pltpu.VMEM((CH, DW), jnp.float32),
            pltpu.VMEM_SHARED((N_ACC, DW), jnp.float32),
        ] + [pltpu.SemaphoreType.DMA] * NBUF,
        compiler_params=pltpu.CompilerParams(use_tc_tiling_on_sc=False),
    )
    def _deg_kernel(dst_hbm, ones_hbm, zeros_hbm, out_hbm, didx, ones_v, acc,
                    *sems):
        c = lax.axis_index("c")
        s = lax.axis_index("s")
        wid = s * NC + c
        pltpu.sync_copy(zeros_hbm.at[pl.ds(s * RPT, RPT)],
                        acc.at[pl.ds(s * RPT, RPT)])
        pltpu.sync_copy(ones_hbm, ones_v)
        pltpu.sync_copy(dst_hbm.at[wid], didx)
        plsc.subcore_barrier()

        def grp(g, carry):
            for b in range(NBUF):
                i = g * NBUF + b

                @pl.when(g > 0)
                def _wait_prev():
                    pltpu.make_async_copy(
                        ones_v, acc.at[didx.at[i]], sems[b]).wait()

                pltpu.async_copy(ones_v, acc.at[didx.at[i]], sems[b], add=True)
            return carry

        lax.fori_loop(0, NGRP, grp, 0)
        for b in range(NBUF):
            pltpu.make_async_copy(ones_v, acc.at[didx.at[b]], sems[b]).wait()
        plsc.subcore_barrier()
        pltpu.sync_copy(acc.at[pl.ds(s * RPT, RPT)],
                        out_hbm.at[c, pl.ds(s * RPT, RPT)])

    return _deg_kernel


@functools.cache
def _make_prop_kernel(width):
    mesh = plsc.VectorSubcoreMesh(core_axis_name="c", subcore_axis_name="s")

    @functools.partial(
        pl.kernel,
        mesh=mesh,
        out_type=jax.ShapeDtypeStruct((NC, N_ACC, width), jnp.float32),
        scratch_types=[
            pltpu.VMEM((CH_PER_W, CH), jnp.int32),
            pltpu.VMEM((CH_PER_W, CH), jnp.int32),
            pltpu.VMEM((NBUF, CH, width), jnp.float32),
            pltpu.VMEM_SHARED((N_ACC, width), jnp.float32),
        ] + [pltpu.SemaphoreType.DMA] * (2 * NBUF),
        compiler_params=pltpu.CompilerParams(use_tc_tiling_on_sc=False),
    )
    def _prop_kernel(table_hbm, src_hbm, dst_hbm, zeros_hbm, out_hbm,
                     sidx, didx, rows, acc, *sems):
        gsem = sems[:NBUF]
        ssem = sems[NBUF:]
        c = lax.axis_index("c")
        s = lax.axis_index("s")
        wid = s * NC + c
        pltpu.sync_copy(zeros_hbm.at[pl.ds(s * RPT, RPT)],
                        acc.at[pl.ds(s * RPT, RPT)])
        pltpu.sync_copy(src_hbm.at[wid], sidx)
        pltpu.sync_copy(dst_hbm.at[wid], didx)
        plsc.subcore_barrier()
        for b in range(PF):
            pltpu.async_copy(table_hbm.at[sidx.at[b]], rows.at[b], gsem[b])

        def grp(g, carry):
            for b in range(NBUF):
                i = g * NBUF + b
                bb = (b + PF) % NBUF
                j = i + PF
                pltpu.make_async_copy(
                    table_hbm.at[sidx.at[i]], rows.at[b], gsem[b]).wait()
                pltpu.async_copy(rows.at[b], acc.at[didx.at[i]], ssem[b],
                                 add=True)

                @pl.when(j < CH_PER_W)
                def _prefetch():
                    @pl.when(i >= NBUF - PF)
                    def _wait_scatter():
                        pltpu.make_async_copy(
                            rows.at[bb], acc.at[didx.at[i]], ssem[bb]).wait()

                    pltpu.async_copy(table_hbm.at[sidx.at[j]], rows.at[bb],
                                     gsem[bb])
            return carry

        lax.fori_loop(0, NGRP, grp, 0)
        for b in range(NBUF):
            pltpu.make_async_copy(
                rows.at[b], acc.at[didx.at[b]], ssem[b]).wait()
        plsc.subcore_barrier()
        pltpu.sync_copy(acc.at[pl.ds(s * RPT, RPT)],
                        out_hbm.at[c, pl.ds(s * RPT, RPT)])

    return _prop_kernel


def _compact_body(logits_ref, x_ref, w1_ref, xc_ref, w1c_ref):
    soft = jax.nn.sigmoid(logits_ref[...])
    a = soft[:, None]
    b = soft[None, :]
    i2 = lax.broadcasted_iota(jnp.int32, (F, F), 0)
    j2 = lax.broadcasted_iota(jnp.int32, (F, F), 1)
    beats = (b > a) | ((b == a) & (j2 < i2))
    rank = jnp.sum(beats.astype(jnp.int32), axis=1)
    sel = rank < K
    before = sel[None, :] & (j2 < i2)
    pos = jnp.sum(before.astype(jnp.int32), axis=1)
    kk = lax.broadcasted_iota(jnp.int32, (F, K), 1)
    P = (sel[:, None] & (pos[:, None] == kk)).astype(jnp.float32)
    xc_ref[...] = jnp.dot(x_ref[...], P, preferred_element_type=jnp.float32)
    w1c_ref[...] = lax.dot_general(
        P, w1_ref[...], (((0,), (0,)), ((), ())),
        preferred_element_type=jnp.float32)


def _scale_body(dc0_ref, dc1_ref, xc_ref, xt_ref, dinv_ref):
    deg = dc0_ref[...] + dc1_ref[...] + 1.0
    dinv = lax.rsqrt(jnp.maximum(deg, 1e-12))
    dinv_ref[...] = dinv
    xt_ref[...] = xc_ref[...] * dinv


def _mid_body(s1a_ref, s1b_ref, xt_ref, dinv_ref, w1c_ref, b1_ref, w2_ref,
              gt_ref):
    dinv = dinv_ref[...]
    xp = (s1a_ref[...] + s1b_ref[...] + xt_ref[...]) * dinv
    h = jnp.dot(xp, w1c_ref[...], preferred_element_type=jnp.float32)
    h = jnp.maximum(h + b1_ref[...], 0.0)
    g = jnp.dot(h, w2_ref[...], preferred_element_type=jnp.float32)
    gt_ref[...] = g * dinv


def _out_body(s2a_ref, s2b_ref, gt_ref, dinv_ref, b2_ref, out_ref):
    out_ref[...] = ((s2a_ref[...] + s2b_ref[...] + gt_ref[...])
                    * dinv_ref[...] + b2_ref[...])


_MB = 2000


def kernel(x, edge_index, feature_logits, W1, b1, W2, b2):
    src = edge_index[0]
    dst = edge_index[1]
    pad = E_PAD - E
    srcp = jnp.concatenate([src, jnp.zeros((pad,), jnp.int32)])
    dstp = jnp.concatenate([dst, jnp.full((pad,), N, jnp.int32)])
    srcp = srcp.reshape(NW, CH_PER_W, CH)
    dstp = dstp.reshape(NW, CH_PER_W, CH)
    zeros_w = jnp.zeros((N_ACC, K), jnp.float32)
    zeros_1 = jnp.zeros((N_ACC, DW), jnp.float32)
    ones_ch = jnp.ones((CH, DW), jnp.float32)

    dc = _make_deg_kernel()(dstp, ones_ch, zeros_1)

    xc, w1c = pl.pallas_call(
        _compact_body,
        out_shape=(jax.ShapeDtypeStruct((N, K), jnp.float32),
                   jax.ShapeDtypeStruct((K, H), jnp.float32)),
    )(feature_logits, x, W1)

    xt, dinv = pl.pallas_call(
        _scale_body,
        out_shape=(jax.ShapeDtypeStruct((N, K), jnp.float32),
                   jax.ShapeDtypeStruct((N, 1), jnp.float32)),
    )(dc[0, :N, :1], dc[1, :N, :1], xc)

    s1 = _make_prop_kernel(K)(xt, srcp, dstp, zeros_w)

    gt = pl.pallas_call(
        _mid_body,
        grid=(N // _MB,),
        in_specs=[
            pl.BlockSpec((_MB, K), lambda i: (i, 0)),
            pl.BlockSpec((_MB, K), lambda i: (i, 0)),
            pl.BlockSpec((_MB, K), lambda i: (i, 0)),
            pl.BlockSpec((_MB, 1), lambda i: (i, 0)),
            pl.BlockSpec((K, H), lambda i: (0, 0)),
            pl.BlockSpec((1, H), lambda i: (0, 0)),
            pl.BlockSpec((H, C), lambda i: (0, 0)),
        ],
        out_specs=pl.BlockSpec((_MB, C), lambda i: (i, 0)),
        out_shape=jax.ShapeDtypeStruct((N, C), jnp.float32),
    )(s1[0, :N], s1[1, :N], xt, dinv, w1c, b1.reshape(1, H), W2)

    s2 = _make_prop_kernel(C)(gt, srcp, dstp, zeros_w)

    out = pl.pallas_call(
        _out_body,
        out_shape=jax.ShapeDtypeStruct((N, C), jnp.float32),
    )(s2[0, :N], s2[1, :N], gt, dinv, b2.reshape(1, C))
    return out


# speedup vs baseline: 35.4196x; 2.2897x over previous
import functools

import jax
import jax.numpy as jnp
from jax import lax
from jax.experimental import pallas as pl
from jax.experimental.pallas import tpu as pltpu
from jax.experimental.pallas import tpu_sc as plsc

N = 10000
E = 320000
F = 128
H = 256
C = 64
K = 64

NC = 2
NS = 16
NW = NC * NS
CH = 128
CHW_D = 80
CHW_P = 160
EPW = CHW_D * CH
E_PAD = NW * EPW
N_ACC = N + 112
RPT = N_ACC // NS
RPN = N // NS
DW = 8
HW = 32
NBUF = 8
PF = 4
NGRP_D = CHW_D // NBUF
NGRP_P = CHW_P // NBUF


@functools.cache
def _make_deg_kernel():
    mesh = plsc.VectorSubcoreMesh(core_axis_name="c", subcore_axis_name="s")

    @functools.partial(
        pl.kernel,
        mesh=mesh,
        out_type=jax.ShapeDtypeStruct((NC, N_ACC, DW), jnp.float32),
        scratch_types=[
            pltpu.VMEM((CHW_D, CH), jnp.int32),
            pltpu.VMEM((CH, DW), jnp.float32),
            pltpu.VMEM_SHARED((N_ACC, DW), jnp.float32),
        ] + [pltpu.SemaphoreType.DMA] * NBUF,
        compiler_params=pltpu.CompilerParams(use_tc_tiling_on_sc=False),
    )
    def _deg_kernel(dst_hbm, ones_hbm, zeros_hbm, out_hbm, didx, ones_v, acc,
                    *sems):
        c = lax.axis_index("c")
        s = lax.axis_index("s")
        wid = s * NC + c
        pltpu.sync_copy(zeros_hbm.at[pl.ds(s * RPT, RPT)],
                        acc.at[pl.ds(s * RPT, RPT)])
        pltpu.sync_copy(ones_hbm, ones_v)
        pltpu.sync_copy(dst_hbm.at[wid], didx)
        plsc.subcore_barrier()

        def grp(g, carry):
            for b in range(NBUF):
                i = g * NBUF + b

                @pl.when(g > 0)
                def _wait_prev():
                    pltpu.make_async_copy(
                        ones_v, acc.at[didx.at[i]], sems[b]).wait()

                pltpu.async_copy(ones_v, acc.at[didx.at[i]], sems[b], add=True)
            return carry

        lax.fori_loop(0, NGRP_D, grp, 0)
        for b in range(NBUF):
            pltpu.make_async_copy(ones_v, acc.at[didx.at[b]], sems[b]).wait()
        plsc.subcore_barrier()
        pltpu.sync_copy(acc.at[pl.ds(s * RPT, RPT)],
                        out_hbm.at[c, pl.ds(s * RPT, RPT)])

    return _deg_kernel


@functools.cache
def _make_prop_kernel():
    mesh = plsc.VectorSubcoreMesh(core_axis_name="c", subcore_axis_name="s")

    @functools.partial(
        pl.kernel,
        mesh=mesh,
        out_type=jax.ShapeDtypeStruct((NC, N_ACC, HW), jnp.float32),
        scratch_types=[
            pltpu.VMEM((CHW_P, CH), jnp.int32),
            pltpu.VMEM((CHW_P, CH), jnp.int32),
            pltpu.VMEM((NBUF, CH, HW), jnp.float32),
            pltpu.VMEM_SHARED((N_ACC, HW), jnp.float32),
            pltpu.VMEM_SHARED((N, HW), jnp.float32),
        ] + [pltpu.SemaphoreType.DMA] * (2 * NBUF),
        compiler_params=pltpu.CompilerParams(use_tc_tiling_on_sc=False),
    )
    def _prop_kernel(table_hbm, src_hbm, dst_hbm, zeros_hbm, out_hbm,
                     sidx, didx, rows, acc, tbl, *sems):
        gsem = sems[:NBUF]
        ssem = sems[NBUF:]
        c = lax.axis_index("c")
        s = lax.axis_index("s")
        pltpu.sync_copy(zeros_hbm.at[pl.ds(s * RPT, RPT)],
                        acc.at[pl.ds(s * RPT, RPT)])
        pltpu.sync_copy(table_hbm.at[c, pl.ds(s * RPN, RPN)],
                        tbl.at[pl.ds(s * RPN, RPN)])
        pltpu.sync_copy(src_hbm.at[s], sidx)
        pltpu.sync_copy(dst_hbm.at[s], didx)
        plsc.subcore_barrier()
        for b in range(PF):
            pltpu.async_copy(tbl.at[sidx.at[b]], rows.at[b], gsem[b])

        def grp(g, carry):
            for b in range(NBUF):
                i = g * NBUF + b
                bb = (b + PF) % NBUF
                j = i + PF
                pltpu.make_async_copy(
                    tbl.at[sidx.at[i]], rows.at[b], gsem[b]).wait()
                pltpu.async_copy(rows.at[b], acc.at[didx.at[i]], ssem[b],
                                 add=True)

                @pl.when(j < CHW_P)
                def _prefetch():
                    @pl.when(i >= NBUF - PF)
                    def _wait_scatter():
                        pltpu.make_async_copy(
                            rows.at[bb], acc.at[didx.at[i]], ssem[bb]).wait()

                    pltpu.async_copy(tbl.at[sidx.at[j]], rows.at[bb],
                                     gsem[bb])
            return carry

        lax.fori_loop(0, NGRP_P, grp, 0)
        for b in range(NBUF):
            pltpu.make_async_copy(
                rows.at[b], acc.at[didx.at[b]], ssem[b]).wait()
        plsc.subcore_barrier()
        pltpu.sync_copy(acc.at[pl.ds(s * RPT, RPT)],
                        out_hbm.at[c, pl.ds(s * RPT, RPT)])

    return _prop_kernel


def _compact_body(logits_ref, x_ref, w1_ref, xc_ref, w1c_ref):
    soft = jax.nn.sigmoid(logits_ref[...])
    a = soft[:, None]
    b = soft[None, :]
    i2 = lax.broadcasted_iota(jnp.int32, (F, F), 0)
    j2 = lax.broadcasted_iota(jnp.int32, (F, F), 1)
    beats = (b > a) | ((b == a) & (j2 < i2))
    rank = jnp.sum(beats.astype(jnp.int32), axis=1)
    sel = rank < K
    before = sel[None, :] & (j2 < i2)
    pos = jnp.sum(before.astype(jnp.int32), axis=1)
    kk = lax.broadcasted_iota(jnp.int32, (F, K), 1)
    P = (sel[:, None] & (pos[:, None] == kk)).astype(jnp.float32)
    xc_ref[...] = jnp.dot(x_ref[...], P, preferred_element_type=jnp.float32)
    w1c_ref[...] = lax.dot_general(
        P, w1_ref[...], (((0,), (0,)), ((), ())),
        preferred_element_type=jnp.float32)


def _scale_body(dc0_ref, dc1_ref, xc_ref, xt_ref, xts_ref, dinv_ref):
    deg = dc0_ref[...] + dc1_ref[...] + 1.0
    dinv = lax.rsqrt(jnp.maximum(deg, 1e-12))
    dinv_ref[...] = dinv
    xt = xc_ref[...] * dinv
    xt_ref[...] = xt
    xts_ref[0] = xt[:, :HW]
    xts_ref[1] = xt[:, HW:]


def _mid_body(s1a_ref, s1b_ref, xt_ref, dinv_ref, w1c_ref, b1_ref, w2_ref,
              gt_ref, gts_ref):
    dinv = dinv_ref[...]
    s1 = jnp.concatenate([s1a_ref[...], s1b_ref[...]], axis=1)
    xp = (s1 + xt_ref[...]) * dinv
    h = jnp.dot(xp, w1c_ref[...], preferred_element_type=jnp.float32)
    h = jnp.maximum(h + b1_ref[...], 0.0)
    g = jnp.dot(h, w2_ref[...], preferred_element_type=jnp.float32)
    gt = g * dinv
    gt_ref[...] = gt
    gts_ref[0] = gt[:, :HW]
    gts_ref[1] = gt[:, HW:]


def _out_body(s2a_ref, s2b_ref, gt_ref, dinv_ref, b2_ref, out_ref):
    s2 = jnp.concatenate([s2a_ref[...], s2b_ref[...]], axis=1)
    out_ref[...] = (s2 + gt_ref[...]) * dinv_ref[...] + b2_ref[...]


_MB = 2000


def kernel(x, edge_index, feature_logits, W1, b1, W2, b2):
    src = edge_index[0]
    dst = edge_index[1]
    pad = E_PAD - E
    srcp = jnp.concatenate([src, jnp.zeros((pad,), jnp.int32)])
    dstp = jnp.concatenate([dst, jnp.full((pad,), N, jnp.int32)])
    dst_d = dstp.reshape(NW, CHW_D, CH)
    src_p = srcp.reshape(NS, CHW_P, CH)
    dst_p = dstp.reshape(NS, CHW_P, CH)
    zeros_h = jnp.zeros((N_ACC, HW), jnp.float32)
    zeros_d = jnp.zeros((N_ACC, DW), jnp.float32)
    ones_ch = jnp.ones((CH, DW), jnp.float32)

    dc = _make_deg_kernel()(dst_d, ones_ch, zeros_d)

    xc, w1c = pl.pallas_call(
        _compact_body,
        out_shape=(jax.ShapeDtypeStruct((N, K), jnp.float32),
                   jax.ShapeDtypeStruct((K, H), jnp.float32)),
    )(feature_logits, x, W1)

    xt, xts, dinv = pl.pallas_call(
        _scale_body,
        out_shape=(jax.ShapeDtypeStruct((N, K), jnp.float32),
                   jax.ShapeDtypeStruct((NC, N, HW), jnp.float32),
                   jax.ShapeDtypeStruct((N, 1), jnp.float32)),
    )(dc[0, :N, :1], dc[1, :N, :1], xc)

    s1 = _make_prop_kernel()(xts, src_p, dst_p, zeros_h)

    gt, gts = pl.pallas_call(
        _mid_body,
        grid=(N // _MB,),
        in_specs=[
            pl.BlockSpec((_MB, HW), lambda i: (i, 0)),
            pl.BlockSpec((_MB, HW), lambda i: (i, 0)),
            pl.BlockSpec((_MB, K), lambda i: (i, 0)),
            pl.BlockSpec((_MB, 1), lambda i: (i, 0)),
            pl.BlockSpec((K, H), lambda i: (0, 0)),
            pl.BlockSpec((1, H), lambda i: (0, 0)),
            pl.BlockSpec((H, C), lambda i: (0, 0)),
        ],
        out_specs=(pl.BlockSpec((_MB, C), lambda i: (i, 0)),
                   pl.BlockSpec((NC, _MB, HW), lambda i: (0, i, 0))),
        out_shape=(jax.ShapeDtypeStruct((N, C), jnp.float32),
                   jax.ShapeDtypeStruct((NC, N, HW), jnp.float32)),
    )(s1[0, :N], s1[1, :N], xt, dinv, w1c, b1.reshape(1, H), W2)

    s2 = _make_prop_kernel()(gts, src_p, dst_p, zeros_h)

    out = pl.pallas_call(
        _out_body,
        out_shape=jax.ShapeDtypeStruct((N, C), jnp.float32),
    )(s2[0, :N], s2[1, :N], gt, dinv, b2.reshape(1, C))
    return out


# speedup vs baseline: 40.0724x; 1.1314x over previous
import functools

import jax
import jax.numpy as jnp
from jax import lax
from jax.experimental import pallas as pl
from jax.experimental.pallas import tpu as pltpu
from jax.experimental.pallas import tpu_sc as plsc

N = 10000
E = 320000
F = 128
H = 256
C = 64
K = 64

NC = 2
NS = 16
NW = NC * NS
CH = 128
CHW_D = 80
CHW_P = 160
EPW = CHW_D * CH
E_PAD = NW * EPW
N_ACC = N + 112
RPT = N_ACC // NS
RPN = N // NS
DW = 8
HW = 32
NBUF = 8
PF = 4
NGRP_D = CHW_D // NBUF
NGRP_P = CHW_P // NBUF


@functools.cache
def _make_deg_kernel():
    mesh = plsc.VectorSubcoreMesh(core_axis_name="c", subcore_axis_name="s")

    @functools.partial(
        pl.kernel,
        mesh=mesh,
        out_type=jax.ShapeDtypeStruct((NC, N_ACC, DW), jnp.float32),
        scratch_types=[
            pltpu.VMEM((CHW_D, CH), jnp.int32),
            pltpu.VMEM((CH, DW), jnp.float32),
            pltpu.VMEM_SHARED((N_ACC, DW), jnp.float32),
        ] + [pltpu.SemaphoreType.DMA] * NBUF,
        compiler_params=pltpu.CompilerParams(use_tc_tiling_on_sc=False),
    )
    def _deg_kernel(dst_hbm, ones_hbm, zeros_hbm, out_hbm, didx, ones_v, acc,
                    *sems):
        c = lax.axis_index("c")
        s = lax.axis_index("s")
        wid = s * NC + c
        pltpu.sync_copy(zeros_hbm.at[pl.ds(s * RPT, RPT)],
                        acc.at[pl.ds(s * RPT, RPT)])
        pltpu.sync_copy(ones_hbm, ones_v)
        pltpu.sync_copy(dst_hbm.at[wid], didx)
        plsc.subcore_barrier()

        def grp(g, carry):
            for b in range(NBUF):
                i = g * NBUF + b

                @pl.when(g > 0)
                def _wait_prev():
                    pltpu.make_async_copy(
                        ones_v, acc.at[didx.at[i]], sems[b]).wait()

                pltpu.async_copy(ones_v, acc.at[didx.at[i]], sems[b], add=True)
            return carry

        lax.fori_loop(0, NGRP_D, grp, 0)
        for b in range(NBUF):
            pltpu.make_async_copy(ones_v, acc.at[didx.at[b]], sems[b]).wait()
        plsc.subcore_barrier()
        pltpu.sync_copy(acc.at[pl.ds(s * RPT, RPT)],
                        out_hbm.at[c, pl.ds(s * RPT, RPT)])

    return _deg_kernel


@functools.cache
def _make_prop_kernel():
    mesh = plsc.VectorSubcoreMesh(core_axis_name="c", subcore_axis_name="s")

    @functools.partial(
        pl.kernel,
        mesh=mesh,
        out_type=jax.ShapeDtypeStruct((NC, N_ACC, HW), jnp.float32),
        scratch_types=[
            pltpu.VMEM((CHW_P, CH), jnp.int32),
            pltpu.VMEM((CHW_P, CH), jnp.int32),
            pltpu.VMEM((NBUF, CH, HW), jnp.float32),
            pltpu.VMEM_SHARED((N_ACC, HW), jnp.float32),
            pltpu.VMEM_SHARED((N, HW), jnp.float32),
        ] + [pltpu.SemaphoreType.DMA] * (2 * NBUF),
        compiler_params=pltpu.CompilerParams(use_tc_tiling_on_sc=False),
    )
    def _prop_kernel(table_hbm, src_hbm, dst_hbm, zeros_hbm, out_hbm,
                     sidx, didx, rows, acc, tbl, *sems):
        gsem = sems[:NBUF]
        ssem = sems[NBUF:]
        c = lax.axis_index("c")
        s = lax.axis_index("s")
        pltpu.sync_copy(zeros_hbm.at[pl.ds(s * RPT, RPT)],
                        acc.at[pl.ds(s * RPT, RPT)])
        pltpu.sync_copy(table_hbm.at[c, pl.ds(s * RPN, RPN)],
                        tbl.at[pl.ds(s * RPN, RPN)])
        pltpu.sync_copy(src_hbm.at[s], sidx)
        pltpu.sync_copy(dst_hbm.at[s], didx)
        plsc.subcore_barrier()
        for b in range(PF):
            pltpu.async_copy(tbl.at[sidx.at[b]], rows.at[b], gsem[b])

        def grp(g, carry):
            for b in range(NBUF):
                i = g * NBUF + b
                bb = (b + PF) % NBUF
                j = i + PF
                pltpu.make_async_copy(
                    tbl.at[sidx.at[i]], rows.at[b], gsem[b]).wait()
                pltpu.async_copy(rows.at[b], acc.at[didx.at[i]], ssem[b],
                                 add=True)

                @pl.when(j < CHW_P)
                def _prefetch():
                    @pl.when(i >= NBUF - PF)
                    def _wait_scatter():
                        pltpu.make_async_copy(
                            rows.at[bb], acc.at[didx.at[i]], ssem[bb]).wait()

                    pltpu.async_copy(tbl.at[sidx.at[j]], rows.at[bb],
                                     gsem[bb])
            return carry

        lax.fori_loop(0, NGRP_P, grp, 0)
        for b in range(NBUF):
            pltpu.make_async_copy(
                rows.at[b], acc.at[didx.at[b]], ssem[b]).wait()
        plsc.subcore_barrier()
        pltpu.sync_copy(acc.at[pl.ds(s * RPT, RPT)],
                        out_hbm.at[c, pl.ds(s * RPT, RPT)])

    return _prop_kernel


def _compact_body(logits_ref, x_ref, w1_ref, xc_ref, w1c_ref):
    soft = jax.nn.sigmoid(logits_ref[...])
    a = soft[:, None]
    b = soft[None, :]
    i2 = lax.broadcasted_iota(jnp.int32, (F, F), 0)
    j2 = lax.broadcasted_iota(jnp.int32, (F, F), 1)
    beats = (b > a) | ((b == a) & (j2 < i2))
    rank = jnp.sum(beats.astype(jnp.int32), axis=1)
    sel = rank < K
    before = sel[None, :] & (j2 < i2)
    pos = jnp.sum(before.astype(jnp.int32), axis=1)
    kk = lax.broadcasted_iota(jnp.int32, (F, K), 1)
    P = (sel[:, None] & (pos[:, None] == kk)).astype(jnp.float32)
    xc_ref[...] = jnp.dot(x_ref[...], P, preferred_element_type=jnp.float32)
    w1c_ref[...] = lax.dot_general(
        P, w1_ref[...], (((0,), (0,)), ((), ())),
        preferred_element_type=jnp.float32)


def _prep_body(ei_ref, src_ref, dst_ref):
    src_ref[pl.ds(0, E)] = ei_ref[0, :]
    dst_ref[pl.ds(0, E)] = ei_ref[1, :]
    src_ref[pl.ds(E, E_PAD - E)] = jnp.zeros((E_PAD - E,), jnp.int32)
    dst_ref[pl.ds(E, E_PAD - E)] = jnp.full((E_PAD - E,), N, jnp.int32)


def _scale_body(dc_ref, xc_ref, xt_ref, xts_ref, dinv_ref):
    dc0 = dc_ref[0, pl.ds(0, N), pl.ds(0, 1)]
    dc1 = dc_ref[1, pl.ds(0, N), pl.ds(0, 1)]
    deg = dc0 + dc1 + 1.0
    dinv = lax.rsqrt(jnp.maximum(deg, 1e-12))
    dinv_ref[...] = dinv
    xt = xc_ref[...] * dinv
    xt_ref[...] = xt
    xts_ref[0] = xt[:, :HW]
    xts_ref[1] = xt[:, HW:]


def _mid_body(s1a_ref, s1b_ref, xt_ref, dinv_ref, w1c_ref, b1_ref, w2_ref,
              gt_ref, gts_ref):
    dinv = dinv_ref[...]
    s1 = jnp.concatenate([s1a_ref[0], s1b_ref[0]], axis=1)
    xp = (s1 + xt_ref[...]) * dinv
    h = jnp.dot(xp, w1c_ref[...], preferred_element_type=jnp.float32)
    h = jnp.maximum(h + b1_ref[...], 0.0)
    g = jnp.dot(h, w2_ref[...], preferred_element_type=jnp.float32)
    gt = g * dinv
    gt_ref[...] = gt
    gts_ref[0] = gt[:, :HW]
    gts_ref[1] = gt[:, HW:]


def _out_body(s2_ref, gt_ref, dinv_ref, b2_ref, out_ref):
    s2 = jnp.concatenate([s2_ref[0, pl.ds(0, N)], s2_ref[1, pl.ds(0, N)]],
                         axis=1)
    out_ref[...] = (s2 + gt_ref[...]) * dinv_ref[...] + b2_ref[...]


_MB = 2000


def kernel(x, edge_index, feature_logits, W1, b1, W2, b2):
    srcp, dstp = pl.pallas_call(
        _prep_body,
        out_shape=(jax.ShapeDtypeStruct((E_PAD,), jnp.int32),
                   jax.ShapeDtypeStruct((E_PAD,), jnp.int32)),
    )(edge_index)
    dst_d = dstp.reshape(NW, CHW_D, CH)
    src_p = srcp.reshape(NS, CHW_P, CH)
    dst_p = dstp.reshape(NS, CHW_P, CH)
    zeros_h = jnp.zeros((N_ACC, HW), jnp.float32)
    zeros_d = jnp.zeros((N_ACC, DW), jnp.float32)
    ones_ch = jnp.ones((CH, DW), jnp.float32)

    dc = _make_deg_kernel()(dst_d, ones_ch, zeros_d)

    xc, w1c = pl.pallas_call(
        _compact_body,
        out_shape=(jax.ShapeDtypeStruct((N, K), jnp.float32),
                   jax.ShapeDtypeStruct((K, H), jnp.float32)),
    )(feature_logits, x, W1)

    xt, xts, dinv = pl.pallas_call(
        _scale_body,
        out_shape=(jax.ShapeDtypeStruct((N, K), jnp.float32),
                   jax.ShapeDtypeStruct((NC, N, HW), jnp.float32),
                   jax.ShapeDtypeStruct((N, 1), jnp.float32)),
    )(dc, xc)

    s1 = _make_prop_kernel()(xts, src_p, dst_p, zeros_h)

    gt, gts = pl.pallas_call(
        _mid_body,
        grid=(N // _MB,),
        in_specs=[
            pl.BlockSpec((1, _MB, HW), lambda i: (0, i, 0)),
            pl.BlockSpec((1, _MB, HW), lambda i: (1, i, 0)),
            pl.BlockSpec((_MB, K), lambda i: (i, 0)),
            pl.BlockSpec((_MB, 1), lambda i: (i, 0)),
            pl.BlockSpec((K, H), lambda i: (0, 0)),
            pl.BlockSpec((1, H), lambda i: (0, 0)),
            pl.BlockSpec((H, C), lambda i: (0, 0)),
        ],
        out_specs=(pl.BlockSpec((_MB, C), lambda i: (i, 0)),
                   pl.BlockSpec((NC, _MB, HW), lambda i: (0, i, 0))),
        out_shape=(jax.ShapeDtypeStruct((N, C), jnp.float32),
                   jax.ShapeDtypeStruct((NC, N, HW), jnp.float32)),
    )(s1, s1, xt, dinv, w1c, b1.reshape(1, H), W2)

    s2 = _make_prop_kernel()(gts, src_p, dst_p, zeros_h)

    out = pl.pallas_call(
        _out_body,
        out_shape=jax.ShapeDtypeStruct((N, C), jnp.float32),
    )(s2, gt, dinv, b2.reshape(1, C))
    return out


# speedup vs baseline: 41.8210x; 1.0436x over previous
import functools

import jax
import jax.numpy as jnp
from jax import lax
from jax.experimental import pallas as pl
from jax.experimental.pallas import tpu as pltpu
from jax.experimental.pallas import tpu_sc as plsc

N = 10000
E = 320000
F = 128
H = 256
C = 64
K = 64

NC = 2
NS = 16
NW = NC * NS
CH = 128
CHW_D = 80
CHW_P = 160
EPW = CHW_D * CH
E_PAD = NW * EPW
N_ACC = N + 112
RPT = N_ACC // NS
RPN = N // NS
DW = 8
HW = 32
NBUF = 8
PF = 4
NGRP_D = CHW_D // NBUF
NGRP_P = CHW_P // NBUF


@functools.cache
def _make_deg_kernel():
    mesh = plsc.VectorSubcoreMesh(core_axis_name="c", subcore_axis_name="s")

    @functools.partial(
        pl.kernel,
        mesh=mesh,
        out_type=jax.ShapeDtypeStruct((NC, N_ACC, DW), jnp.float32),
        scratch_types=[
            pltpu.VMEM((CHW_D, CH), jnp.int32),
            pltpu.VMEM((CH, DW), jnp.float32),
            pltpu.VMEM_SHARED((N_ACC, DW), jnp.float32),
        ] + [pltpu.SemaphoreType.DMA] * NBUF,
        compiler_params=pltpu.CompilerParams(use_tc_tiling_on_sc=False),
    )
    def _deg_kernel(dst_hbm, ones_hbm, zeros_hbm, out_hbm, didx, ones_v, acc,
                    *sems):
        c = lax.axis_index("c")
        s = lax.axis_index("s")
        wid = s * NC + c
        pltpu.sync_copy(zeros_hbm.at[pl.ds(s * RPT, RPT)],
                        acc.at[pl.ds(s * RPT, RPT)])
        pltpu.sync_copy(ones_hbm, ones_v)
        pltpu.sync_copy(dst_hbm.at[wid], didx)
        plsc.subcore_barrier()

        def grp(g, carry):
            for b in range(NBUF):
                i = g * NBUF + b

                @pl.when(g > 0)
                def _wait_prev():
                    pltpu.make_async_copy(
                        ones_v, acc.at[didx.at[i]], sems[b]).wait()

                pltpu.async_copy(ones_v, acc.at[didx.at[i]], sems[b], add=True)
            return carry

        lax.fori_loop(0, NGRP_D, grp, 0)
        for b in range(NBUF):
            pltpu.make_async_copy(ones_v, acc.at[didx.at[b]], sems[b]).wait()
        plsc.subcore_barrier()
        pltpu.sync_copy(acc.at[pl.ds(s * RPT, RPT)],
                        out_hbm.at[c, pl.ds(s * RPT, RPT)])

    return _deg_kernel


@functools.cache
def _make_prop_kernel():
    mesh = plsc.VectorSubcoreMesh(core_axis_name="c", subcore_axis_name="s")

    @functools.partial(
        pl.kernel,
        mesh=mesh,
        out_type=jax.ShapeDtypeStruct((NC, N_ACC, HW), jnp.float32),
        scratch_types=[
            pltpu.VMEM((CHW_P, CH), jnp.int32),
            pltpu.VMEM((CHW_P, CH), jnp.int32),
            pltpu.VMEM((NBUF, CH, HW), jnp.float32),
            pltpu.VMEM_SHARED((N_ACC, HW), jnp.float32),
            pltpu.VMEM_SHARED((N, HW), jnp.float32),
        ] + [pltpu.SemaphoreType.DMA] * (2 * NBUF),
        compiler_params=pltpu.CompilerParams(use_tc_tiling_on_sc=False),
    )
    def _prop_kernel(table_hbm, src_hbm, dst_hbm, zeros_hbm, out_hbm,
                     sidx, didx, rows, acc, tbl, *sems):
        gsem = sems[:NBUF]
        ssem = sems[NBUF:]
        c = lax.axis_index("c")
        s = lax.axis_index("s")
        pltpu.sync_copy(zeros_hbm.at[pl.ds(s * RPT, RPT)],
                        acc.at[pl.ds(s * RPT, RPT)])
        pltpu.sync_copy(table_hbm.at[pl.ds(s * RPN, RPN), pl.ds(c * HW, HW)],
                        tbl.at[pl.ds(s * RPN, RPN)])
        pltpu.sync_copy(src_hbm.at[s], sidx)
        pltpu.sync_copy(dst_hbm.at[s], didx)
        plsc.subcore_barrier()
        for b in range(PF):
            pltpu.async_copy(tbl.at[sidx.at[b]], rows.at[b], gsem[b])

        def grp(g, carry):
            for b in range(NBUF):
                i = g * NBUF + b
                bb = (b + PF) % NBUF
                j = i + PF
                pltpu.make_async_copy(
                    tbl.at[sidx.at[i]], rows.at[b], gsem[b]).wait()
                pltpu.async_copy(rows.at[b], acc.at[didx.at[i]], ssem[b],
                                 add=True)

                @pl.when(j < CHW_P)
                def _prefetch():
                    @pl.when(i >= NBUF - PF)
                    def _wait_scatter():
                        pltpu.make_async_copy(
                            rows.at[bb], acc.at[didx.at[i]], ssem[bb]).wait()

                    pltpu.async_copy(tbl.at[sidx.at[j]], rows.at[bb],
                                     gsem[bb])
            return carry

        lax.fori_loop(0, NGRP_P, grp, 0)
        for b in range(NBUF):
            pltpu.make_async_copy(
                rows.at[b], acc.at[didx.at[b]], ssem[b]).wait()
        plsc.subcore_barrier()
        pltpu.sync_copy(acc.at[pl.ds(s * RPT, RPT)],
                        out_hbm.at[c, pl.ds(s * RPT, RPT)])

    return _prop_kernel


def _compact_body(logits_ref, x_ref, w1_ref, xc_ref, w1c_ref):
    soft = jax.nn.sigmoid(logits_ref[...])
    a = soft[:, None]
    b = soft[None, :]
    i2 = lax.broadcasted_iota(jnp.int32, (F, F), 0)
    j2 = lax.broadcasted_iota(jnp.int32, (F, F), 1)
    beats = (b > a) | ((b == a) & (j2 < i2))
    rank = jnp.sum(beats.astype(jnp.int32), axis=1)
    sel = rank < K
    before = sel[None, :] & (j2 < i2)
    pos = jnp.sum(before.astype(jnp.int32), axis=1)
    kk = lax.broadcasted_iota(jnp.int32, (F, K), 1)
    P = (sel[:, None] & (pos[:, None] == kk)).astype(jnp.float32)
    xc_ref[...] = jnp.dot(x_ref[...], P, preferred_element_type=jnp.float32)
    w1c_ref[...] = lax.dot_general(
        P, w1_ref[...], (((0,), (0,)), ((), ())),
        preferred_element_type=jnp.float32)


def _prep_body(ei_ref, src_ref, dst_ref):
    src_ref[pl.ds(0, E)] = ei_ref[0, :]
    dst_ref[pl.ds(0, E)] = ei_ref[1, :]
    src_ref[pl.ds(E, E_PAD - E)] = jnp.zeros((E_PAD - E,), jnp.int32)
    dst_ref[pl.ds(E, E_PAD - E)] = jnp.full((E_PAD - E,), N, jnp.int32)


def _scale_body(dc_ref, xc_ref, xt_ref, dinv_ref):
    dc0 = dc_ref[0, pl.ds(0, N), pl.ds(0, 1)]
    dc1 = dc_ref[1, pl.ds(0, N), pl.ds(0, 1)]
    deg = dc0 + dc1 + 1.0
    dinv = lax.rsqrt(jnp.maximum(deg, 1e-12))
    dinv_ref[...] = dinv
    xt_ref[...] = xc_ref[...] * dinv


def _mid_body(s1a_ref, s1b_ref, xt_ref, dinv_ref, w1c_ref, b1_ref, w2_ref,
              gt_ref):
    dinv = dinv_ref[...]
    s1 = jnp.concatenate([s1a_ref[0], s1b_ref[0]], axis=1)
    xp = (s1 + xt_ref[...]) * dinv
    h = jnp.dot(xp, w1c_ref[...], preferred_element_type=jnp.float32)
    h = jnp.maximum(h + b1_ref[...], 0.0)
    g = jnp.dot(h, w2_ref[...], preferred_element_type=jnp.float32)
    gt_ref[...] = g * dinv


def _out_body(s2_ref, gt_ref, dinv_ref, b2_ref, out_ref):
    s2 = jnp.concatenate([s2_ref[0, pl.ds(0, N)], s2_ref[1, pl.ds(0, N)]],
                         axis=1)
    out_ref[...] = (s2 + gt_ref[...]) * dinv_ref[...] + b2_ref[...]


_MB = 2000


def kernel(x, edge_index, feature_logits, W1, b1, W2, b2):
    srcp, dstp = pl.pallas_call(
        _prep_body,
        out_shape=(jax.ShapeDtypeStruct((E_PAD,), jnp.int32),
                   jax.ShapeDtypeStruct((E_PAD,), jnp.int32)),
    )(edge_index)
    dst_d = dstp.reshape(NW, CHW_D, CH)
    src_p = srcp.reshape(NS, CHW_P, CH)
    dst_p = dstp.reshape(NS, CHW_P, CH)
    zeros_h = jnp.zeros((N_ACC, HW), jnp.float32)
    zeros_d = jnp.zeros((N_ACC, DW), jnp.float32)
    ones_ch = jnp.ones((CH, DW), jnp.float32)

    dc = _make_deg_kernel()(dst_d, ones_ch, zeros_d)

    xc, w1c = pl.pallas_call(
        _compact_body,
        out_shape=(jax.ShapeDtypeStruct((N, K), jnp.float32),
                   jax.ShapeDtypeStruct((K, H), jnp.float32)),
    )(feature_logits, x, W1)

    xt, dinv = pl.pallas_call(
        _scale_body,
        out_shape=(jax.ShapeDtypeStruct((N, K), jnp.float32),
                   jax.ShapeDtypeStruct((N, 1), jnp.float32)),
    )(dc, xc)

    s1 = _make_prop_kernel()(xt, src_p, dst_p, zeros_h)

    gt = pl.pallas_call(
        _mid_body,
        grid=(N // _MB,),
        in_specs=[
            pl.BlockSpec((1, _MB, HW), lambda i: (0, i, 0)),
            pl.BlockSpec((1, _MB, HW), lambda i: (1, i, 0)),
            pl.BlockSpec((_MB, K), lambda i: (i, 0)),
            pl.BlockSpec((_MB, 1), lambda i: (i, 0)),
            pl.BlockSpec((K, H), lambda i: (0, 0)),
            pl.BlockSpec((1, H), lambda i: (0, 0)),
            pl.BlockSpec((H, C), lambda i: (0, 0)),
        ],
        out_specs=pl.BlockSpec((_MB, C), lambda i: (i, 0)),
        out_shape=jax.ShapeDtypeStruct((N, C), jnp.float32),
    )(s1, s1, xt, dinv, w1c, b1.reshape(1, H), W2)

    s2 = _make_prop_kernel()(gt, src_p, dst_p, zeros_h)

    out = pl.pallas_call(
        _out_body,
        out_shape=jax.ShapeDtypeStruct((N, C), jnp.float32),
    )(s2, gt, dinv, b2.reshape(1, C))
    return out
